# baseline (speedup 1.0000x reference)
import functools

import jax
import jax.numpy as jnp
import numpy as np
from jax import lax
from jax.experimental import pallas as pl
from jax.experimental.pallas import tpu as pltpu
from jax.experimental.pallas import tpu_sc as plsc

N = 10000
E = 320000
D_IN = 128
D_OUT = 256
DH = 128
D_EDGE = 16

NS = 16
EPT = E // NS
C = 80
NCHUNK = EPT // C
NPT = 640
NPT_STRIDE = 624

_S3 = 3.0 ** 0.5
_S5 = 5.0 ** 0.5
_S15 = 15.0 ** 0.5

_PERM_H = np.array(
    [(g * 16 + r) if r < 16 else (64 + g * 16 + r - 16)
     for g in range(DH // 32) for r in range(32)], dtype=np.int32)


def _pack_bf16_pairs(a, b):
    ua = lax.bitcast_convert_type(
        a.astype(jnp.bfloat16).astype(jnp.float32), jnp.uint32)
    ub = lax.bitcast_convert_type(
        b.astype(jnp.bfloat16).astype(jnp.float32), jnp.uint32)
    word = (lax.shift_right_logical(ua, jnp.uint32(16))
            | (ub & jnp.uint32(0xFFFF0000)))
    return lax.bitcast_convert_type(word, jnp.int32)


def _node_mlp_body(x_ref, w1_ref, b1_ref, w2_ref, b2_ref, lo_ref, hi_ref):
    h = jax.nn.silu(
        jnp.dot(x_ref[...], w1_ref[...], preferred_element_type=jnp.float32)
        + b1_ref[...]
    )
    o = jnp.dot(h, w2_ref[...], preferred_element_type=jnp.float32) + b2_ref[...]
    lo_ref[...] = _pack_bf16_pairs(o[:, :64], o[:, 64:128])
    hi_ref[...] = _pack_bf16_pairs(o[:, 128:192], o[:, 192:])


def _node_mlp(x, Wn1, bn1, Wn2, bn2):
    return pl.pallas_call(
        _node_mlp_body,
        out_shape=(
            jax.ShapeDtypeStruct((N, DH // 2), jnp.int32),
            jax.ShapeDtypeStruct((N, DH // 2), jnp.int32),
        ),
    )(x, Wn1, bn1.reshape(1, -1), Wn2, bn2.reshape(1, -1))


CC = 400
EPW = E // 32


def _sc_sh_body(posx, posy, posz, rowi, coli,
                shT_o, cnt2_o,
                px_v, py_v, pz_v, ridx_v, cidx_v, sh_v, hist_v,
                hred_v, cdr_v, hist_sh, sem):
    cid = lax.axis_index("c")
    sid = lax.axis_index("s")
    wid = cid * NS + sid
    base = wid * EPW

    pltpu.sync_copy(posx, px_v)
    pltpu.sync_copy(posy, py_v)
    pltpu.sync_copy(posz, pz_v)

    zeros16 = jnp.zeros((16,), jnp.float32)
    ones16 = jnp.ones((16,), jnp.float32)

    def zh(i, _):
        hist_v[pl.ds(i * 16, 16)] = zeros16
        return 0

    lax.fori_loop(0, N // 16, zh, 0)
    for k in range(9, 16):
        for j in range(CC // 16):
            sh_v[k, pl.ds(j * 16, 16)] = zeros16

    def chunk(s, _):
        e0 = base + s * CC
        pltpu.sync_copy(rowi.at[pl.ds(e0, CC)], ridx_v)
        pltpu.sync_copy(coli.at[pl.ds(e0, CC)], cidx_v)

        def inner(j, _):
            sl = pl.ds(j * 16, 16)
            ri = ridx_v[sl]
            ci = cidx_v[sl]
            rx = plsc.load_gather(px_v, [ri]) - plsc.load_gather(px_v, [ci])
            ry = plsc.load_gather(py_v, [ri]) - plsc.load_gather(py_v, [ci])
            rz = plsc.load_gather(pz_v, [ri]) - plsc.load_gather(pz_v, [ci])
            r2 = jnp.maximum(rx * rx + ry * ry + rz * rz, 1e-24)
            ib = plsc.bitcast(r2, jnp.int32)
            ib = 0x5F3759DF - lax.shift_right_logical(ib, 1)
            y = plsc.bitcast(ib, jnp.float32)
            y = y * (1.5 - 0.5 * r2 * y * y)
            y = y * (1.5 - 0.5 * r2 * y * y)
            y = y * (1.5 - 0.5 * r2 * y * y)
            t = r2 * y + 1e-8
            z = y * (2.0 - t * y)
            z = z * (2.0 - t * z)
            dx = rx * z
            dy = ry * z
            dz = rz * z
            sh_v[0, sl] = ones16
            sh_v[1, sl] = _S3 * dx
            sh_v[2, sl] = _S3 * dy
            sh_v[3, sl] = _S3 * dz
            sh_v[4, sl] = _S15 * dx * dy
            sh_v[5, sl] = _S15 * dy * dz
            sh_v[6, sl] = (_S5 / 2.0) * (3.0 * dz * dz - 1.0)
            sh_v[7, sl] = _S15 * dx * dz
            sh_v[8, sl] = (_S15 / 2.0) * (dx * dx - dy * dy)
            plsc.addupdate_scatter(hist_v, [ci], ones16)
            return 0

        lax.fori_loop(0, CC // 16, inner, 0)
        pltpu.sync_copy(sh_v, shT_o.at[:, pl.ds(e0, CC)])
        return 0

    lax.fori_loop(0, EPW // CC, chunk, 0)

    pltpu.sync_copy(hist_v, hist_sh.at[sid])
    plsc.subcore_barrier()
    r0 = sid * NPT_STRIDE
    pltpu.sync_copy(hist_sh.at[:, pl.ds(r0, NPT)], hred_v)

    def rsum(i, _):
        sl = pl.ds(i * 16, 16)
        acc = hred_v[0, sl]
        for r in range(1, NS):
            acc = acc + hred_v[r, sl]
        cdr_v[sl] = acc
        return 0

    lax.fori_loop(0, NPT // 16, rsum, 0)
    pltpu.sync_copy(cdr_v, cnt2_o.at[cid, pl.ds(r0, NPT)])


def _sc_sh(posx, posy, posz, row, col):
    f = pl.kernel(
        _sc_sh_body,
        out_type=(
            jax.ShapeDtypeStruct((16, E), jnp.float32),
            jax.ShapeDtypeStruct((2, N), jnp.float32),
        ),
        mesh=plsc.VectorSubcoreMesh(core_axis_name="c", subcore_axis_name="s"),
        compiler_params=pltpu.CompilerParams(use_tc_tiling_on_sc=False, needs_layout_passes=False),
        scratch_types=[
            pltpu.VMEM((N,), jnp.float32),
            pltpu.VMEM((N,), jnp.float32),
            pltpu.VMEM((N,), jnp.float32),
            pltpu.VMEM((CC,), jnp.int32),
            pltpu.VMEM((CC,), jnp.int32),
            pltpu.VMEM((16, CC), jnp.float32),
            pltpu.VMEM((N,), jnp.float32),
            pltpu.VMEM((NS, NPT), jnp.float32),
            pltpu.VMEM((NPT,), jnp.float32),
            pltpu.VMEM_SHARED((NS, N), jnp.float32),
            pltpu.SemaphoreType.DMA,
        ],
    )
    return f(posx, posy, posz, row, col)


def _edge_body(shT_ref, ea_ref, w1a_ref, w1b_ref, b1_ref, w2_ref, b2_ref,
               lo_ref, hi_ref):
    h = lax.dot_general(shT_ref[...], w1a_ref[...],
                        dimension_numbers=(((0,), (0,)), ((), ())),
                        preferred_element_type=jnp.float32)
    h = h + jnp.dot(ea_ref[...], w1b_ref[...], preferred_element_type=jnp.float32)
    h = jax.nn.silu(h + b1_ref[...])
    em = jnp.dot(h, w2_ref[...], preferred_element_type=jnp.float32) + b2_ref[...]
    lo_ref[...] = _pack_bf16_pairs(em[:, :64], em[:, 64:128])
    hi_ref[...] = _pack_bf16_pairs(em[:, 128:192], em[:, 192:])


def _edge_msg(shT, edge_attr, We1, be1, We2, be2):
    B = 2560
    grid = E // B
    return pl.pallas_call(
        _edge_body,
        grid=(grid,),
        in_specs=[
            pl.BlockSpec((16, B), lambda i: (0, i)),
            pl.BlockSpec((B, D_EDGE), lambda i: (i, 0)),
            pl.BlockSpec((16, D_OUT), lambda i: (0, 0)),
            pl.BlockSpec((D_EDGE, D_OUT), lambda i: (0, 0)),
            pl.BlockSpec((1, D_OUT), lambda i: (0, 0)),
            pl.BlockSpec((D_OUT, D_OUT), lambda i: (0, 0)),
            pl.BlockSpec((1, D_OUT), lambda i: (0, 0)),
        ],
        out_specs=(
            pl.BlockSpec((B, DH // 2), lambda i: (i, 0)),
            pl.BlockSpec((B, DH // 2), lambda i: (i, 0)),
        ),
        out_shape=(
            jax.ShapeDtypeStruct((E, DH // 2), jnp.int32),
            jax.ShapeDtypeStruct((E, DH // 2), jnp.int32),
        ),
    )(shT, edge_attr,
      jnp.pad(We1[:9], ((0, 7), (0, 0))), We1[9:], be1.reshape(1, -1),
      We2, be2.reshape(1, -1))


SUP = 10


def _sc_body(emlo, emhi, xtlo, xthi, rowi2, coli2,
             agglo, agghi,
             em0, em1, xt0, xt1, msg0, msg1, ridx_b, cidx_b,
             agg_sh, esem0, esem1, gsem0, gsem1, ssem0, ssem1):
    cid = lax.axis_index("c")
    sid = lax.axis_index("s")
    r0 = sid * NPT_STRIDE

    zeros16 = jnp.zeros((16,), jnp.float32)

    def zrow(i, _):
        for j in range(DH // 16):
            msg0[i, pl.ds(j * 16, 16)] = zeros16
        return 0

    lax.fori_loop(0, C, zrow, 0)

    for j in range(NPT // C):
        pltpu.sync_copy(msg0, agg_sh.at[pl.ds(r0 + j * C, C)])
    plsc.subcore_barrier()

    def run(em_ref, xt_ref, agg_out):
        ems = (em0, em1)
        xts = (xt0, xt1)
        msgs = (msg0, msg1)
        esems = (esem0, esem1)
        gsems = (gsem0, gsem1)
        ssems = (ssem0, ssem1)
        base_chunk = sid * NCHUNK

        def super_chunk(s, _):
            c0 = base_chunk + s * SUP
            pltpu.sync_copy(rowi2.at[pl.ds(c0 * C, SUP * C)], ridx_b)
            pltpu.sync_copy(coli2.at[pl.ds(c0 * C, SUP * C)], cidx_b)

            def issue(j):
                b = j & 1
                e = pltpu.async_copy(
                    em_ref.at[pl.ds((c0 + j) * C, C)], ems[b], esems[b])
                g = pltpu.async_copy(
                    xt_ref.at[ridx_b.at[pl.ds(j * C, C)]], xts[b], gsems[b])
                return e, g

            pend = issue(0)
            spend = None
            for j in range(SUP):
                b = j & 1
                nxt = issue(j + 1) if j + 1 < SUP else None
                pend[0].wait()
                pend[1].wait()

                def mrow(i, _):
                    sixteen = jnp.int32(16)
                    himask = jnp.int32(-65536)
                    for q in range(DH // 32):
                        sq = (i, pl.ds(q * 16, 16))
                        wa = ems[b][sq]
                        wb = xts[b][sq]
                        alo = plsc.bitcast(lax.shift_left(wa, sixteen),
                                           jnp.float32)
                        blo = plsc.bitcast(lax.shift_left(wb, sixteen),
                                           jnp.float32)
                        ahi = plsc.bitcast(wa & himask, jnp.float32)
                        bhi = plsc.bitcast(wb & himask, jnp.float32)
                        msgs[b][i, pl.ds(q * 32, 16)] = alo * blo
                        msgs[b][i, pl.ds(q * 32 + 16, 16)] = ahi * bhi
                    return 0

                lax.fori_loop(0, C, mrow, 0)
                if spend is not None:
                    spend.wait()
                spend = pltpu.async_copy(
                    msgs[b], agg_sh.at[cidx_b.at[pl.ds(j * C, C)]],
                    ssems[b], add=True)
                pend = nxt
            spend.wait()
            return 0

        lax.fori_loop(0, NCHUNK // SUP, super_chunk, 0)
        plsc.subcore_barrier()
        for j in range(NPT // C):
            pltpu.sync_copy(agg_sh.at[pl.ds(r0 + j * C, C)], msg0)
            pltpu.sync_copy(msg0, agg_out.at[pl.ds(r0 + j * C, C)])

    @pl.when(cid == 0)
    def _():
        run(emlo, xtlo, agglo)

    @pl.when(cid == 1)
    def _():
        run(emhi, xthi, agghi)


def _sc_aggregate(em_lo, em_hi, xt_lo, xt_hi, row2, col2):
    f = pl.kernel(
        _sc_body,
        out_type=(
            jax.ShapeDtypeStruct((N, DH), jnp.float32),
            jax.ShapeDtypeStruct((N, DH), jnp.float32),
        ),
        mesh=plsc.VectorSubcoreMesh(core_axis_name="c", subcore_axis_name="s"),
        compiler_params=pltpu.CompilerParams(use_tc_tiling_on_sc=False, needs_layout_passes=False),
        scratch_types=[
            pltpu.VMEM((C, DH // 2), jnp.int32),
            pltpu.VMEM((C, DH // 2), jnp.int32),
            pltpu.VMEM((C, DH // 2), jnp.int32),
            pltpu.VMEM((C, DH // 2), jnp.int32),
            pltpu.VMEM((C, DH), jnp.float32),
            pltpu.VMEM((C, DH), jnp.float32),
            pltpu.VMEM((SUP * C,), jnp.int32),
            pltpu.VMEM((SUP * C,), jnp.int32),
            pltpu.VMEM_SHARED((N, DH), jnp.float32),
            pltpu.SemaphoreType.DMA,
            pltpu.SemaphoreType.DMA,
            pltpu.SemaphoreType.DMA,
            pltpu.SemaphoreType.DMA,
            pltpu.SemaphoreType.DMA,
            pltpu.SemaphoreType.DMA,
        ],
    )
    return f(em_lo, em_hi, xt_lo, xt_hi, row2, col2)


def _out_body(agglo_ref, agghi_ref, cnt_ref, x_ref, w1a_ref, w1b_ref,
              w1c_ref, b1_ref, w2_ref, b2_ref, g_ref, bt_ref, out_ref):
    inv_cnt = 1.0 / jnp.maximum(cnt_ref[...], 1.0)
    alo = agglo_ref[...] * inv_cnt
    ahi = agghi_ref[...] * inv_cnt
    h = jax.nn.silu(
        jnp.dot(alo, w1a_ref[...], preferred_element_type=jnp.float32)
        + jnp.dot(ahi, w1b_ref[...], preferred_element_type=jnp.float32)
        + jnp.dot(x_ref[...], w1c_ref[...], preferred_element_type=jnp.float32)
        + b1_ref[...]
    )
    h = jnp.dot(h, w2_ref[...], preferred_element_type=jnp.float32) + b2_ref[...]
    mu = jnp.mean(h, axis=0, keepdims=True)
    var = jnp.mean((h - mu) ** 2, axis=0, keepdims=True)
    out_ref[...] = (h - mu) * lax.rsqrt(var + 1e-5) * g_ref[...] + bt_ref[...]


def _out_mlp(agg_lo, agg_hi, cnt, x, Wo1, bo1, Wo2, bo2, gamma, beta):
    return pl.pallas_call(
        _out_body,
        out_shape=jax.ShapeDtypeStruct((N, D_OUT), jnp.float32),
    )(agg_lo, agg_hi, cnt, x, Wo1[:DH][_PERM_H], Wo1[DH:D_OUT][_PERM_H],
      Wo1[D_OUT:],
      bo1.reshape(1, -1), Wo2, bo2.reshape(1, -1), gamma.reshape(1, -1),
      beta.reshape(1, -1))


def kernel(x, edge_index, edge_attr, pos, Wn1, bn1, Wn2, bn2, We1, be1, We2,
           be2, Wo1, bo1, Wo2, bo2, gamma, beta):
    row = edge_index[0]
    col = edge_index[1]
    xt_lo, xt_hi = _node_mlp(x, Wn1, bn1, Wn2, bn2)
    shT, cnt2 = _sc_sh(pos[:, 0], pos[:, 1], pos[:, 2], row, col)
    em_lo, em_hi = _edge_msg(shT, edge_attr, We1, be1, We2, be2)
    agg_lo, agg_hi = _sc_aggregate(em_lo, em_hi, xt_lo, xt_hi, row, col)
    cnt = (cnt2[0] + cnt2[1]).reshape(N, 1)
    return _out_mlp(agg_lo, agg_hi, cnt, x, Wo1, bo1, Wo2, bo2, gamma, beta)


# speedup vs baseline: 1.3277x; 1.3277x over previous
import functools

import jax
import jax.numpy as jnp
import numpy as np
from jax import lax
from jax.experimental import pallas as pl
from jax.experimental.pallas import tpu as pltpu
from jax.experimental.pallas import tpu_sc as plsc

N = 10000
E = 320000
D_IN = 128
D_OUT = 256
DH = 128
D_EDGE = 16

NS = 16
EPT = E // NS
C = 80
NCHUNK = EPT // C
NPT = 640
NPT_STRIDE = 624

_S3 = 3.0 ** 0.5
_S5 = 5.0 ** 0.5
_S15 = 15.0 ** 0.5

_PERM_H = np.array(
    [(g * 16 + r) if r < 16 else (64 + g * 16 + r - 16)
     for g in range(DH // 32) for r in range(32)], dtype=np.int32)


def _pack_bf16_pairs(a, b):
    ua = lax.bitcast_convert_type(
        a.astype(jnp.bfloat16).astype(jnp.float32), jnp.uint32)
    ub = lax.bitcast_convert_type(
        b.astype(jnp.bfloat16).astype(jnp.float32), jnp.uint32)
    word = (lax.shift_right_logical(ua, jnp.uint32(16))
            | (ub & jnp.uint32(0xFFFF0000)))
    return lax.bitcast_convert_type(word, jnp.int32)


def _node_mlp_body(x_ref, w1_ref, b1_ref, w2_ref, b2_ref, lo_ref, hi_ref):
    h = jax.nn.silu(
        jnp.dot(x_ref[...], w1_ref[...], preferred_element_type=jnp.float32)
        + b1_ref[...]
    )
    o = jnp.dot(h, w2_ref[...], preferred_element_type=jnp.float32) + b2_ref[...]
    lo_ref[...] = _pack_bf16_pairs(o[:, :64], o[:, 64:128])
    hi_ref[...] = _pack_bf16_pairs(o[:, 128:192], o[:, 192:])


def _node_mlp(x, Wn1, bn1, Wn2, bn2):
    return pl.pallas_call(
        _node_mlp_body,
        out_shape=(
            jax.ShapeDtypeStruct((N, DH // 2), jnp.int32),
            jax.ShapeDtypeStruct((N, DH // 2), jnp.int32),
        ),
    )(x, Wn1, bn1.reshape(1, -1), Wn2, bn2.reshape(1, -1))


CC = 400
EPW = E // 32


def _sc_sh_body(posx, posy, posz, rowi, coli,
                shT_o, cnt2_o,
                px_v, py_v, pz_v, ridx_v, cidx_v, sh_v, hist_v,
                hred_v, cdr_v, hist_sh, sem):
    cid = lax.axis_index("c")
    sid = lax.axis_index("s")
    wid = cid * NS + sid
    base = wid * EPW

    pltpu.sync_copy(posx, px_v)
    pltpu.sync_copy(posy, py_v)
    pltpu.sync_copy(posz, pz_v)

    zeros16 = jnp.zeros((16,), jnp.float32)
    ones16 = jnp.ones((16,), jnp.float32)

    def zh(i, _):
        hist_v[pl.ds(i * 16, 16)] = zeros16
        return 0

    lax.fori_loop(0, N // 16, zh, 0)
    for k in range(9, 16):
        for j in range(CC // 16):
            sh_v[k, pl.ds(j * 16, 16)] = zeros16

    def chunk(s, _):
        e0 = base + s * CC
        pltpu.sync_copy(rowi.at[pl.ds(e0, CC)], ridx_v)
        pltpu.sync_copy(coli.at[pl.ds(e0, CC)], cidx_v)

        def inner(j, _):
            sl = pl.ds(j * 16, 16)
            ri = ridx_v[sl]
            ci = cidx_v[sl]
            rx = plsc.load_gather(px_v, [ri]) - plsc.load_gather(px_v, [ci])
            ry = plsc.load_gather(py_v, [ri]) - plsc.load_gather(py_v, [ci])
            rz = plsc.load_gather(pz_v, [ri]) - plsc.load_gather(pz_v, [ci])
            r2 = jnp.maximum(rx * rx + ry * ry + rz * rz, 1e-24)
            ib = plsc.bitcast(r2, jnp.int32)
            ib = 0x5F3759DF - lax.shift_right_logical(ib, 1)
            y = plsc.bitcast(ib, jnp.float32)
            y = y * (1.5 - 0.5 * r2 * y * y)
            y = y * (1.5 - 0.5 * r2 * y * y)
            y = y * (1.5 - 0.5 * r2 * y * y)
            t = r2 * y + 1e-8
            z = y * (2.0 - t * y)
            z = z * (2.0 - t * z)
            dx = rx * z
            dy = ry * z
            dz = rz * z
            sh_v[0, sl] = ones16
            sh_v[1, sl] = _S3 * dx
            sh_v[2, sl] = _S3 * dy
            sh_v[3, sl] = _S3 * dz
            sh_v[4, sl] = _S15 * dx * dy
            sh_v[5, sl] = _S15 * dy * dz
            sh_v[6, sl] = (_S5 / 2.0) * (3.0 * dz * dz - 1.0)
            sh_v[7, sl] = _S15 * dx * dz
            sh_v[8, sl] = (_S15 / 2.0) * (dx * dx - dy * dy)
            plsc.addupdate_scatter(hist_v, [ci], ones16)
            return 0

        lax.fori_loop(0, CC // 16, inner, 0)
        pltpu.sync_copy(sh_v, shT_o.at[:, pl.ds(e0, CC)])
        return 0

    lax.fori_loop(0, EPW // CC, chunk, 0)

    pltpu.sync_copy(hist_v, hist_sh.at[sid])
    plsc.subcore_barrier()
    r0 = sid * NPT_STRIDE
    pltpu.sync_copy(hist_sh.at[:, pl.ds(r0, NPT)], hred_v)

    def rsum(i, _):
        sl = pl.ds(i * 16, 16)
        acc = hred_v[0, sl]
        for r in range(1, NS):
            acc = acc + hred_v[r, sl]
        cdr_v[sl] = acc
        return 0

    lax.fori_loop(0, NPT // 16, rsum, 0)
    pltpu.sync_copy(cdr_v, cnt2_o.at[cid, pl.ds(r0, NPT)])


def _sc_sh(posx, posy, posz, row, col):
    f = pl.kernel(
        _sc_sh_body,
        out_type=(
            jax.ShapeDtypeStruct((16, E), jnp.float32),
            jax.ShapeDtypeStruct((2, N), jnp.float32),
        ),
        mesh=plsc.VectorSubcoreMesh(core_axis_name="c", subcore_axis_name="s"),
        compiler_params=pltpu.CompilerParams(use_tc_tiling_on_sc=False, needs_layout_passes=False),
        scratch_types=[
            pltpu.VMEM((N,), jnp.float32),
            pltpu.VMEM((N,), jnp.float32),
            pltpu.VMEM((N,), jnp.float32),
            pltpu.VMEM((CC,), jnp.int32),
            pltpu.VMEM((CC,), jnp.int32),
            pltpu.VMEM((16, CC), jnp.float32),
            pltpu.VMEM((N,), jnp.float32),
            pltpu.VMEM((NS, NPT), jnp.float32),
            pltpu.VMEM((NPT,), jnp.float32),
            pltpu.VMEM_SHARED((NS, N), jnp.float32),
            pltpu.SemaphoreType.DMA,
        ],
    )
    return f(posx, posy, posz, row, col)


def _edge_body(shT_ref, ea_ref, w1a_ref, w1b_ref, b1_ref, w2_ref, b2_ref,
               lo_ref, hi_ref):
    h = lax.dot_general(shT_ref[...], w1a_ref[...],
                        dimension_numbers=(((0,), (0,)), ((), ())),
                        preferred_element_type=jnp.float32)
    h = h + jnp.dot(ea_ref[...], w1b_ref[...], preferred_element_type=jnp.float32)
    h = jax.nn.silu(h + b1_ref[...])
    em = jnp.dot(h, w2_ref[...], preferred_element_type=jnp.float32) + b2_ref[...]
    lo_ref[...] = em[:, :DH]
    hi_ref[...] = em[:, DH:]


def _edge_msg(shT, edge_attr, We1, be1, We2, be2):
    B = 2560
    grid = E // B
    return pl.pallas_call(
        _edge_body,
        grid=(grid,),
        in_specs=[
            pl.BlockSpec((16, B), lambda i: (0, i)),
            pl.BlockSpec((B, D_EDGE), lambda i: (i, 0)),
            pl.BlockSpec((16, D_OUT), lambda i: (0, 0)),
            pl.BlockSpec((D_EDGE, D_OUT), lambda i: (0, 0)),
            pl.BlockSpec((1, D_OUT), lambda i: (0, 0)),
            pl.BlockSpec((D_OUT, D_OUT), lambda i: (0, 0)),
            pl.BlockSpec((1, D_OUT), lambda i: (0, 0)),
        ],
        out_specs=(
            pl.BlockSpec((B, DH), lambda i: (i, 0)),
            pl.BlockSpec((B, DH), lambda i: (i, 0)),
        ),
        out_shape=(
            jax.ShapeDtypeStruct((E, DH), jnp.float32),
            jax.ShapeDtypeStruct((E, DH), jnp.float32),
        ),
    )(shT, edge_attr,
      jnp.pad(We1[:9], ((0, 7), (0, 0))), We1[9:], be1.reshape(1, -1),
      We2, be2.reshape(1, -1))


SUP = 10


def _sc_body(emlo, emhi, xtlo, xthi, rowi2, coli2,
             agglo, agghi,
             em0, em1, em2, xt0, xt1, ridx_b, cidx_b,
             agg_sh, esem0, esem1, esem2, gsem0, gsem1, ssem):
    cid = lax.axis_index("c")
    sid = lax.axis_index("s")
    r0 = sid * NPT_STRIDE

    zeros16 = jnp.zeros((16,), jnp.float32)

    def zrow(i, _):
        for j in range(DH // 16):
            em0[i, pl.ds(j * 16, 16)] = zeros16
        return 0

    lax.fori_loop(0, C, zrow, 0)

    for j in range(NPT // C):
        pltpu.sync_copy(em0, agg_sh.at[pl.ds(r0 + j * C, C)])
    plsc.subcore_barrier()

    def run(em_ref, xt_ref, agg_out):
        ems = (em0, em1, em2)
        xts = (xt0, xt1)
        esems = (esem0, esem1, esem2)
        gsems = (gsem0, gsem1)
        base_chunk = sid * NCHUNK

        def super_chunk(s, _):
            c0 = base_chunk + s * SUP
            pltpu.sync_copy(rowi2.at[pl.ds(c0 * C, SUP * C)], ridx_b)
            pltpu.sync_copy(coli2.at[pl.ds(c0 * C, SUP * C)], cidx_b)

            def issue(j):
                e = pltpu.async_copy(
                    em_ref.at[pl.ds((c0 + j) * C, C)], ems[j % 3],
                    esems[j % 3])
                g = pltpu.async_copy(
                    xt_ref.at[ridx_b.at[pl.ds(j * C, C)]], xts[j % 2],
                    gsems[j % 2])
                return e, g

            pend = [issue(0), issue(1)]
            spend = None
            for j in range(SUP):
                b3 = j % 3
                b2 = j % 2
                pend[j][0].wait()
                pend[j][1].wait()

                def mrow(i, _):
                    sixteen = jnp.int32(16)
                    himask = jnp.int32(-65536)
                    for q in range(DH // 32):
                        w = xts[b2][i, pl.ds(q * 16, 16)]
                        lo = plsc.bitcast(lax.shift_left(w, sixteen),
                                          jnp.float32)
                        hi = plsc.bitcast(w & himask, jnp.float32)
                        slo = (i, pl.ds(q * 16, 16))
                        shi = (i, pl.ds(64 + q * 16, 16))
                        ems[b3][slo] = ems[b3][slo] * lo
                        ems[b3][shi] = ems[b3][shi] * hi
                    return 0

                lax.fori_loop(0, C, mrow, 0)
                if spend is not None:
                    spend.wait()
                spend = pltpu.async_copy(
                    ems[b3], agg_sh.at[cidx_b.at[pl.ds(j * C, C)]],
                    ssem, add=True)
                if j + 2 < SUP:
                    pend.append(issue(j + 2))
                else:
                    pend.append(None)
            spend.wait()
            return 0

        lax.fori_loop(0, NCHUNK // SUP, super_chunk, 0)
        plsc.subcore_barrier()
        for j in range(NPT // C):
            pltpu.sync_copy(agg_sh.at[pl.ds(r0 + j * C, C)], em0)
            pltpu.sync_copy(em0, agg_out.at[pl.ds(r0 + j * C, C)])

    @pl.when(cid == 0)
    def _():
        run(emlo, xtlo, agglo)

    @pl.when(cid == 1)
    def _():
        run(emhi, xthi, agghi)


def _sc_aggregate(em_lo, em_hi, xt_lo, xt_hi, row2, col2):
    f = pl.kernel(
        _sc_body,
        out_type=(
            jax.ShapeDtypeStruct((N, DH), jnp.float32),
            jax.ShapeDtypeStruct((N, DH), jnp.float32),
        ),
        mesh=plsc.VectorSubcoreMesh(core_axis_name="c", subcore_axis_name="s"),
        compiler_params=pltpu.CompilerParams(use_tc_tiling_on_sc=False, needs_layout_passes=False),
        scratch_types=[
            pltpu.VMEM((C, DH), jnp.float32),
            pltpu.VMEM((C, DH), jnp.float32),
            pltpu.VMEM((C, DH), jnp.float32),
            pltpu.VMEM((C, DH // 2), jnp.int32),
            pltpu.VMEM((C, DH // 2), jnp.int32),
            pltpu.VMEM((SUP * C,), jnp.int32),
            pltpu.VMEM((SUP * C,), jnp.int32),
            pltpu.VMEM_SHARED((N, DH), jnp.float32),
            pltpu.SemaphoreType.DMA,
            pltpu.SemaphoreType.DMA,
            pltpu.SemaphoreType.DMA,
            pltpu.SemaphoreType.DMA,
            pltpu.SemaphoreType.DMA,
            pltpu.SemaphoreType.DMA,
        ],
    )
    return f(em_lo, em_hi, xt_lo, xt_hi, row2, col2)


def _out_body(agglo_ref, agghi_ref, cnt_ref, x_ref, w1a_ref, w1b_ref,
              w1c_ref, b1_ref, w2_ref, b2_ref, g_ref, bt_ref, out_ref):
    inv_cnt = 1.0 / jnp.maximum(cnt_ref[...], 1.0)
    alo = agglo_ref[...] * inv_cnt
    ahi = agghi_ref[...] * inv_cnt
    h = jax.nn.silu(
        jnp.dot(alo, w1a_ref[...], preferred_element_type=jnp.float32)
        + jnp.dot(ahi, w1b_ref[...], preferred_element_type=jnp.float32)
        + jnp.dot(x_ref[...], w1c_ref[...], preferred_element_type=jnp.float32)
        + b1_ref[...]
    )
    h = jnp.dot(h, w2_ref[...], preferred_element_type=jnp.float32) + b2_ref[...]
    mu = jnp.mean(h, axis=0, keepdims=True)
    var = jnp.mean((h - mu) ** 2, axis=0, keepdims=True)
    out_ref[...] = (h - mu) * lax.rsqrt(var + 1e-5) * g_ref[...] + bt_ref[...]


def _out_mlp(agg_lo, agg_hi, cnt, x, Wo1, bo1, Wo2, bo2, gamma, beta):
    return pl.pallas_call(
        _out_body,
        out_shape=jax.ShapeDtypeStruct((N, D_OUT), jnp.float32),
    )(agg_lo, agg_hi, cnt, x, Wo1[:DH], Wo1[DH:D_OUT], Wo1[D_OUT:],
      bo1.reshape(1, -1), Wo2, bo2.reshape(1, -1), gamma.reshape(1, -1),
      beta.reshape(1, -1))


def kernel(x, edge_index, edge_attr, pos, Wn1, bn1, Wn2, bn2, We1, be1, We2,
           be2, Wo1, bo1, Wo2, bo2, gamma, beta):
    row = edge_index[0]
    col = edge_index[1]
    xt_lo, xt_hi = _node_mlp(x, Wn1, bn1, Wn2, bn2)
    shT, cnt2 = _sc_sh(pos[:, 0], pos[:, 1], pos[:, 2], row, col)
    em_lo, em_hi = _edge_msg(shT, edge_attr, We1, be1, We2, be2)
    agg_lo, agg_hi = _sc_aggregate(em_lo, em_hi, xt_lo, xt_hi, row, col)
    cnt = (cnt2[0] + cnt2[1]).reshape(N, 1)
    return _out_mlp(agg_lo, agg_hi, cnt, x, Wo1, bo1, Wo2, bo2, gamma, beta)
